# Initial kernel scaffold; baseline (speedup 1.0000x reference)
import jax
import jax.numpy as jnp
from jax.experimental import pallas as pl


def kernel(x, edge_index, batch, W_first, b_first, g_first, be_first, W_lin0, b_lin0, W_conv, b_conv, g_conv, be_conv, W_br0, b_br0, W_br1, b_br1, g_mlp0, be_mlp0, W_mlp0, b_mlp0, g_mlp1, be_mlp1, W_mlp1, b_mlp1):
    raise NotImplementedError("write your pallas kernel here")


# speedup vs baseline: 4.4776x; 4.4776x over previous
import functools

import jax
import jax.numpy as jnp
from jax import lax
from jax.experimental import pallas as pl
from jax.experimental.pallas import tpu as pltpu
from jax.experimental.pallas import tpu_sc as plsc

_EPS = 1e-5


def _tdot(a, b):
    return lax.dot_general(a, b, (((0,), (0,)), ((), ())),
                           preferred_element_type=jnp.float32)


def _sigmoid(x):
    return 1.0 / (1.0 + jnp.exp(-x))


def _ka_body(N, Nb, x_ref, wf_ref, bf_ref, gf_ref, bef_ref, wl_ref, bl_ref,
             feat_ref, ns0_ref, y1_ref, st_ref):
    p = pl.program_id(0)
    i = pl.program_id(1)

    @pl.when((p == 0) & (i == 0))
    def _():
        st_ref[...] = jnp.zeros_like(st_ref)

    @pl.when(p == 0)
    def _():
        y = jnp.dot(x_ref[...], wf_ref[...],
                    preferred_element_type=jnp.float32) + bf_ref[...]
        y1_ref[pl.ds(i * Nb, Nb), :] = y
        st_ref[0:1, :] += jnp.sum(y, axis=0, keepdims=True)
        st_ref[1:2, :] += jnp.sum(y * y, axis=0, keepdims=True)

    @pl.when(p == 1)
    def _():
        mean = st_ref[0:1, :] / N
        var = st_ref[1:2, :] / N - mean * mean
        scale = gf_ref[...] * lax.rsqrt(var + _EPS)
        shift = bef_ref[...] - mean * scale
        y = y1_ref[pl.ds(i * Nb, Nb), :]
        f = jnp.maximum(y * scale + shift, 0.0)
        feat_ref[...] = f
        ns0_ref[...] = jnp.dot(f, wl_ref[...],
                               preferred_element_type=jnp.float32) + bl_ref[...]


def _ka_call(x, wf, bf, gf, bef, wl, bl):
    N, D = x.shape
    H = wf.shape[1]
    T = wl.shape[1]
    Nb = 1000
    NB = N // Nb
    grid = (2, NB)
    return pl.pallas_call(
        functools.partial(_ka_body, N, Nb),
        grid=grid,
        in_specs=[
            pl.BlockSpec((Nb, D), lambda p, i: (jnp.where(p == 0, i, 0), 0)),
            pl.BlockSpec((D, H), lambda p, i: (0, 0)),
            pl.BlockSpec((1, H), lambda p, i: (0, 0)),
            pl.BlockSpec((1, H), lambda p, i: (0, 0)),
            pl.BlockSpec((1, H), lambda p, i: (0, 0)),
            pl.BlockSpec((H, T), lambda p, i: (0, 0)),
            pl.BlockSpec((1, T), lambda p, i: (0, 0)),
        ],
        out_specs=[
            pl.BlockSpec((Nb, H), lambda p, i: (jnp.where(p == 1, i, 0), 0)),
            pl.BlockSpec((Nb, T), lambda p, i: (jnp.where(p == 1, i, 0), 0)),
        ],
        out_shape=[
            jax.ShapeDtypeStruct((N, H), jnp.float32),
            jax.ShapeDtypeStruct((N, T), jnp.float32),
        ],
        scratch_shapes=[
            pltpu.VMEM((N, H), jnp.float32),
            pltpu.VMEM((2, H), jnp.float32),
        ],
    )(x, wf, bf, gf, bef, wl, bl)


def _sc_agg(f2, src2, dst2, zeros, Np):
    J = src2.shape[0]
    SCHUNKS = J // 8
    RPS = Np // 16
    mesh = plsc.VectorSubcoreMesh(core_axis_name="c", subcore_axis_name="s",
                                  num_cores=2, num_subcores=16)

    @functools.partial(
        pl.kernel,
        mesh=mesh,
        out_type=[
            jax.ShapeDtypeStruct((Np, 128), jnp.float32),
            jax.ShapeDtypeStruct((Np, 128), jnp.float32),
        ],
        scratch_types=[
            pltpu.VMEM((8, 128), jnp.int32),
            pltpu.VMEM((8, 128), jnp.int32),
            pltpu.VMEM((128, 128), jnp.float32),
            pltpu.VMEM_SHARED((Np, 128), jnp.float32),
            pltpu.SemaphoreType.DMA,
        ],
    )
    def k(f2_hbm, src_hbm, dst_hbm, z_hbm, alo_hbm, ahi_hbm,
          srcv, dstv, rows, acc, sem):
        c = lax.axis_index("c")
        s = lax.axis_index("s")
        row0 = pl.multiple_of(s * RPS, 8)
        pltpu.sync_copy(z_hbm, acc.at[pl.ds(row0, RPS)])
        plsc.subcore_barrier()

        nt = (SCHUNKS - s + 15) // 16

        def body(t, carry):
            j8 = s + t * 16
            base = pl.multiple_of(j8 * 8, 8)
            pltpu.sync_copy(src_hbm.at[pl.ds(base, 8)], srcv)
            pltpu.sync_copy(dst_hbm.at[pl.ds(base, 8)], dstv)
            for r in range(8):
                for kk in range(8):
                    sl = (r, pl.ds(kk * 16, 16))
                    srcv[sl] = srcv[sl] * 2 + c
                pltpu.async_copy(f2_hbm.at[srcv.at[r]], rows, sem).wait()
                pltpu.sync_copy(rows, acc.at[dstv.at[r]], add=True)
            return carry

        lax.fori_loop(0, nt, body, 0)
        plsc.subcore_barrier()

        @pl.when(c == 0)
        def _():
            pltpu.sync_copy(acc.at[pl.ds(row0, RPS)],
                            alo_hbm.at[pl.ds(row0, RPS)])

        @pl.when(c == 1)
        def _():
            pltpu.sync_copy(acc.at[pl.ds(row0, RPS)],
                            ahi_hbm.at[pl.ds(row0, RPS)])

    return k(f2, src2, dst2, zeros)


def _kb_body(N, Nb, G,
             feat_ref, alo_ref, ahi_ref, ns0_ref, b3_ref,
             wc_ref, bc_ref, gc_ref, bec_ref, wbr_ref, bbr_ref,
             gm_ref, bem_ref, wm2_ref, bm2_ref,
             node_ref, wsi_ref,
             z_ref, nall_ref, st_ref, p0_ref, pbr_ref, cnt_ref):
    p = pl.program_id(0)
    i = pl.program_id(1)

    @pl.when((p == 0) & (i == 0))
    def _():
        st_ref[...] = jnp.zeros_like(st_ref)
        p0_ref[...] = jnp.zeros_like(p0_ref)
        pbr_ref[...] = jnp.zeros_like(pbr_ref)
        cnt_ref[...] = jnp.zeros_like(cnt_ref)

    @pl.when(p == 0)
    def _():
        agg = jnp.concatenate([alo_ref[...], ahi_ref[...]], axis=1)
        u = feat_ref[...] + agg
        z = jnp.dot(u, wc_ref[...],
                    preferred_element_type=jnp.float32) + bc_ref[...]
        z_ref[pl.ds(i * Nb, Nb), :] = z
        st_ref[0:1, :] += jnp.sum(z, axis=0, keepdims=True)
        st_ref[1:2, :] += jnp.sum(z * z, axis=0, keepdims=True)

    @pl.when(p == 1)
    def _():
        mean = st_ref[0:1, :] / N
        var = st_ref[1:2, :] / N - mean * mean
        zscale = gc_ref[...] * lax.rsqrt(var + _EPS)
        zshift = bec_ref[...] - mean * zscale
        z = z_ref[pl.ds(i * Nb, Nb), :]
        br = jnp.maximum(z * zscale + zshift, 0.0)
        ns = jnp.dot(br, wbr_ref[...],
                     preferred_element_type=jnp.float32) + bbr_ref[...]
        n0 = ns0_ref[...]
        node = ns + jnp.concatenate([n0, n0], axis=1)
        nall_ref[pl.ds(i * Nb, Nb), :] = node
        st_ref[2:3, :] += jnp.sum(node, axis=0, keepdims=True)
        st_ref[3:4, :] += jnp.sum(node * node, axis=0, keepdims=True)
        bb = b3_ref[0, 0, :]
        gids = lax.broadcasted_iota(jnp.int32, (1, G), 1)
        oh = (bb[:, None] == gids).astype(jnp.float32)
        p0_ref[...] += _tdot(oh, n0)
        pbr_ref[...] += _tdot(oh, ns)
        cnt_ref[...] += _tdot(oh, jnp.ones_like(n0))

    @pl.when(p == 2)
    def _():
        nm = st_ref[2:3, :] / N
        nv = st_ref[3:4, :] / N - nm * nm
        nscale = gm_ref[...] * lax.rsqrt(nv + _EPS)
        nshift = bem_ref[...] - nm * nscale
        node = nall_ref[pl.ds(i * Nb, Nb), :]
        nhat = node * nscale + nshift
        out = _sigmoid(jnp.dot(nhat, wm2_ref[...],
                               preferred_element_type=jnp.float32) + bm2_ref[...])
        node_ref[...] = out

        @pl.when(i == 0)
        def _():
            cc = jnp.maximum(cnt_ref[...], 1.0)
            wsi_pred = p0_ref[...] / cc
            cc2 = jnp.concatenate([cc, cc], axis=1)
            wsi = jnp.concatenate([wsi_pred, wsi_pred], axis=1) \
                + pbr_ref[...] / cc2
            wm = jnp.mean(wsi, axis=0, keepdims=True)
            wv = jnp.mean((wsi - wm) ** 2, axis=0, keepdims=True)
            wh = (wsi - wm) / jnp.sqrt(wv + _EPS) * gm_ref[...] + bem_ref[...]
            wout = _sigmoid(jnp.dot(wh, wm2_ref[...],
                                    preferred_element_type=jnp.float32)
                            + bm2_ref[...])
            wsi_ref[...] = wout


def _kb_call(feat, alo, ahi, ns0, batch3, wc, bc, gc, bec, wbr, bbr,
             gm, bem, wm2, bm2):
    N, H = feat.shape
    T = ns0.shape[1]
    G = 8
    Nb = 1000
    NB = N // Nb
    grid = (3, NB)
    w0 = lambda p, i: (0, 0)
    return pl.pallas_call(
        functools.partial(_kb_body, N, Nb, G),
        grid=grid,
        in_specs=[
            pl.BlockSpec((Nb, H), lambda p, i: (jnp.where(p == 0, i, 0), 0)),
            pl.BlockSpec((Nb, 128), lambda p, i: (jnp.where(p == 0, i, 0), 0)),
            pl.BlockSpec((Nb, 128), lambda p, i: (jnp.where(p == 0, i, 0), 0)),
            pl.BlockSpec((Nb, T), lambda p, i: (jnp.where(p == 1, i, 0), 0)),
            pl.BlockSpec((1, 1, Nb),
                         lambda p, i: (jnp.where(p == 1, i, 0), 0, 0)),
            pl.BlockSpec((H, H), w0),
            pl.BlockSpec((1, H), w0),
            pl.BlockSpec((1, H), w0),
            pl.BlockSpec((1, H), w0),
            pl.BlockSpec((H, 2 * T), w0),
            pl.BlockSpec((1, 2 * T), w0),
            pl.BlockSpec((1, 2 * T), w0),
            pl.BlockSpec((1, 2 * T), w0),
            pl.BlockSpec((2 * T, 2), w0),
            pl.BlockSpec((1, 2), w0),
        ],
        out_specs=[
            pl.BlockSpec((Nb, 2), lambda p, i: (jnp.where(p == 2, i, 0), 0)),
            pl.BlockSpec((G, 2), lambda p, i: (0, 0)),
        ],
        out_shape=[
            jax.ShapeDtypeStruct((N, 2), jnp.float32),
            jax.ShapeDtypeStruct((G, 2), jnp.float32),
        ],
        scratch_shapes=[
            pltpu.VMEM((N, H), jnp.float32),
            pltpu.VMEM((N, 2 * T), jnp.float32),
            pltpu.VMEM((4, 2 * T), jnp.float32),
            pltpu.VMEM((G, T), jnp.float32),
            pltpu.VMEM((G, 2 * T), jnp.float32),
            pltpu.VMEM((G, T), jnp.float32),
        ],
    )(feat, alo, ahi, ns0, batch3, wc, bc, gc, bec, wbr, bbr,
      gm, bem, wm2, bm2)


def kernel(x, edge_index, batch, W_first, b_first, g_first, be_first,
           W_lin0, b_lin0, W_conv, b_conv, g_conv, be_conv,
           W_br0, b_br0, W_br1, b_br1, g_mlp0, be_mlp0, W_mlp0, b_mlp0,
           g_mlp1, be_mlp1, W_mlp1, b_mlp1):
    N, D = x.shape
    H = W_first.shape[1]
    T = W_lin0.shape[1]
    E = edge_index.shape[1]
    Nb = 1000
    NB = N // Nb

    feat, ns0 = _ka_call(
        x, W_first, b_first.reshape(1, H), g_first.reshape(1, H),
        be_first.reshape(1, H), W_lin0, b_lin0.reshape(1, T))

    f2 = feat.reshape(2 * N, 128)
    J = E // 128
    Jp = ((J + 7) // 8) * 8
    Np = 16 * 640
    src2 = jnp.concatenate(
        [edge_index[0].reshape(J, 128),
         jnp.zeros((Jp - J, 128), jnp.int32)], axis=0)
    dst2 = jnp.concatenate(
        [edge_index[1].reshape(J, 128),
         jnp.full((Jp - J, 128), N, jnp.int32)], axis=0)
    zeros = jnp.zeros((Np // 16, 128), jnp.float32)
    alo_p, ahi_p = _sc_agg(f2, src2, dst2, zeros, Np)
    alo, ahi = alo_p[:N], ahi_p[:N]

    wbr = jnp.concatenate([W_br0, W_br1], axis=1)
    bbr = jnp.concatenate([b_br0, b_br1]).reshape(1, 2 * T)
    gm = jnp.concatenate([g_mlp0, g_mlp1]).reshape(1, 2 * T)
    bem = jnp.concatenate([be_mlp0, be_mlp1]).reshape(1, 2 * T)
    wm2 = jnp.zeros((2 * T, 2), jnp.float32)
    wm2 = wm2.at[:T, 0:1].set(W_mlp0).at[T:, 1:2].set(W_mlp1)
    bm2 = jnp.concatenate([b_mlp0, b_mlp1]).reshape(1, 2)
    batch3 = batch.reshape(NB, 1, Nb)

    node01, wsi01 = _kb_call(
        feat, alo, ahi, ns0, batch3, W_conv, b_conv.reshape(1, H),
        g_conv.reshape(1, H), be_conv.reshape(1, H), wbr, bbr,
        gm, bem, wm2, bm2)

    return (wsi01[:, 0:1], node01[:, 0:1], wsi01[:, 1:2], node01[:, 1:2])
